# baseline (speedup 1.0000x reference)
import functools

import jax
import jax.numpy as jnp
from jax import lax
from jax.experimental import pallas as pl
from jax.experimental.pallas import tpu as pltpu
from jax.experimental.pallas import tpu_sc as plsc

N = 1024
NTR = 819
NTE = 205
NTR_PAD = 832
NTE_PAD = 208
LANES = 16


def _fwd_kernel(seq1_ref, adj_ref, Wstru_ref, bstru_ref, Wgat_ref,
                attsrc_ref, attdst_ref, bgat_ref, Wa1_ref, ba1_ref,
                Wa2_ref, ba2_ref, score_ref):
    f32 = jnp.float32
    seq1 = seq1_ref[...]
    adj = adj_ref[...]

    h = jnp.maximum(
        lax.dot_general(seq1, Wstru_ref[...], (((1,), (1,)), ((), ())),
                        preferred_element_type=f32) + bstru_ref[...], 0.0)
    xw = lax.dot_general(h, Wgat_ref[...], (((1,), (1,)), ((), ())),
                         preferred_element_type=f32)

    a_src = jnp.dot(xw, attsrc_ref[...], preferred_element_type=f32)
    a_dst = lax.dot_general(attdst_ref[...], xw, (((0,), (1,)), ((), ())),
                            preferred_element_type=f32)

    z = a_src + a_dst
    e = jnp.where(z >= 0.0, z, 0.2 * z)

    rows = lax.broadcasted_iota(jnp.int32, (N, N), 0)
    cols = lax.broadcasted_iota(jnp.int32, (N, N), 1)
    cnt = adj + jnp.where(rows == cols, 1.0, 0.0)
    mask = cnt > 0.0

    m = jnp.max(jnp.where(mask, e, -1e30), axis=0, keepdims=True)
    w = cnt * jnp.exp(jnp.where(mask, e - m, -60.0))

    num = lax.dot_general(w, xw, (((0,), (0,)), ((), ())),
                          preferred_element_type=f32)
    ones = jnp.ones((N, 1), f32)
    den = lax.dot_general(w, ones, (((0,), (0,)), ((), ())),
                          preferred_element_type=f32)
    emb = num / (den + 1e-16) + bgat_ref[...]

    x = jnp.maximum(
        lax.dot_general(seq1, Wa1_ref[...], (((1,), (1,)), ((), ())),
                        preferred_element_type=f32) + ba1_ref[...], 0.0)
    x_ = lax.dot_general(x, Wa2_ref[...], (((1,), (1,)), ((), ())),
                         preferred_element_type=f32) + ba2_ref[...]
    da = seq1 - x_
    attr_err = jnp.sqrt(jnp.sum(da * da, axis=1, keepdims=True))

    p = lax.dot_general(emb, emb, (((1,), (1,)), ((), ())),
                        preferred_element_type=f32)
    s = jax.nn.sigmoid(p)
    ds = adj - s
    stru_err = jnp.sqrt(jnp.sum(ds * ds, axis=1, keepdims=True))

    score_ref[...] = 0.5 * attr_err + 0.5 * stru_err


def _gather_body(score_hbm, idxtr_hbm, idxte_hbm, loss_hbm, test_hbm,
                 score_v, idxtr_v, idxte_v, loss_v, test_v):
    is_worker = jnp.logical_and(lax.axis_index("c") == 0,
                                lax.axis_index("s") == 0)

    @pl.when(is_worker)
    def _():
        pltpu.sync_copy(score_hbm, score_v)
        pltpu.sync_copy(idxtr_hbm, idxtr_v)
        pltpu.sync_copy(idxte_hbm, idxte_v)

        lane = lax.iota(jnp.int32, LANES)
        acc = jnp.zeros((LANES,), jnp.float32)
        for i in range(NTR_PAD // LANES):
            idx = idxtr_v[pl.ds(i * LANES, LANES)]
            g = plsc.load_gather(score_v, [idx])
            acc = acc + jnp.where(lane + i * LANES < NTR, g, 0.0)
        loss_v[...] = jnp.broadcast_to(jnp.sum(acc) * jnp.float32(1.0 / NTR),
                                       (LANES,))

        for i in range(NTE_PAD // LANES):
            idx = idxte_v[pl.ds(i * LANES, LANES)]
            test_v[pl.ds(i * LANES, LANES)] = plsc.load_gather(score_v, [idx])

        pltpu.sync_copy(loss_v, loss_hbm)
        pltpu.sync_copy(test_v, test_hbm)


def _sc_gather(score, idxtr, idxte):
    return pl.kernel(
        _gather_body,
        out_type=(
            jax.ShapeDtypeStruct((LANES,), jnp.float32),
            jax.ShapeDtypeStruct((NTE_PAD,), jnp.float32),
        ),
        mesh=plsc.VectorSubcoreMesh(core_axis_name="c",
                                    subcore_axis_name="s"),
        compiler_params=pltpu.CompilerParams(needs_layout_passes=False),
        scratch_types=[
            pltpu.VMEM((N,), jnp.float32),
            pltpu.VMEM((NTR_PAD,), jnp.int32),
            pltpu.VMEM((NTE_PAD,), jnp.int32),
            pltpu.VMEM((LANES,), jnp.float32),
            pltpu.VMEM((NTE_PAD,), jnp.float32),
        ],
    )(score, idxtr, idxte)


def kernel(seq1, adj, idx_train, idx_test, W_stru, b_stru, W_gat, att_src,
           att_dst, b_gat, W_a1, b_a1, W_a2, b_a2):
    f32 = jnp.float32
    seq1 = jnp.asarray(seq1, f32).reshape(N, 128)
    adj = jnp.asarray(adj, f32).reshape(N, N)
    idxtr = jnp.zeros((NTR_PAD,), jnp.int32).at[:NTR].set(
        jnp.asarray(idx_train, jnp.int32).reshape(NTR))
    idxte = jnp.zeros((NTE_PAD,), jnp.int32).at[:NTE].set(
        jnp.asarray(idx_test, jnp.int32).reshape(NTE))

    score = pl.pallas_call(
        _fwd_kernel,
        out_shape=jax.ShapeDtypeStruct((N, 1), f32),
    )(seq1, adj,
      W_stru, b_stru.reshape(1, 64),
      W_gat, att_src.reshape(128, 1), att_dst.reshape(128, 1),
      b_gat.reshape(1, 128),
      W_a1, b_a1.reshape(1, 64),
      W_a2, b_a2.reshape(1, 128))

    loss16, test = _sc_gather(score.reshape(N), idxtr, idxte)
    return (loss16[0].reshape(()), test[:NTE])
